# baseline (speedup 1.0000x reference)
import functools

import jax
import jax.numpy as jnp
from jax import lax
from jax.experimental import pallas as pl
from jax.experimental.pallas import tpu as pltpu
from jax.experimental.pallas import tpu_sc as plsc

_NUM_QUERIES = 300
_EMBED_DIM = 256
_FLAT = _NUM_QUERIES * _EMBED_DIM


@functools.lru_cache(maxsize=None)
def _build(batch: int):
    info = plsc.get_sparse_core_info()
    num_workers = info.num_cores * info.num_subcores
    chunk = _FLAT // num_workers
    assert _FLAT % num_workers == 0 and chunk % 8 == 0

    mesh = plsc.VectorSubcoreMesh(core_axis_name="c", subcore_axis_name="s")

    @functools.partial(
        pl.kernel,
        mesh=mesh,
        out_type=jax.ShapeDtypeStruct((batch * _FLAT,), jnp.float32),
        scratch_types=[
            pltpu.VMEM((chunk,), jnp.float32),
            pltpu.SemaphoreType.DMA,
        ],
    )
    def tile_broadcast(table_hbm, out_hbm, buf, sem):
        wid = lax.axis_index("s") * info.num_cores + lax.axis_index("c")
        base = wid * chunk
        pltpu.sync_copy(table_hbm.at[pl.ds(base, chunk)], buf)
        writes = []
        for b in range(batch):
            writes.append(
                pltpu.async_copy(buf, out_hbm.at[pl.ds(b * _FLAT + base, chunk)], sem)
            )
        for w in writes:
            w.wait()

    return tile_broadcast


def kernel(x, table):
    batch = x.shape[0]
    out_flat = _build(batch)(table.reshape(_FLAT))
    return out_flat.reshape(batch, _NUM_QUERIES, _EMBED_DIM)


# speedup vs baseline: 1.0034x; 1.0034x over previous
import functools

import jax
import jax.numpy as jnp
from jax import lax
from jax.experimental import pallas as pl
from jax.experimental.pallas import tpu as pltpu
from jax.experimental.pallas import tpu_sc as plsc

_NUM_QUERIES = 300
_EMBED_DIM = 256
_FLAT = _NUM_QUERIES * _EMBED_DIM


@functools.lru_cache(maxsize=None)
def _build(batch: int):
    info = plsc.get_sparse_core_info()
    num_workers = info.num_cores * info.num_subcores
    chunk = _FLAT // num_workers
    assert _FLAT % num_workers == 0 and chunk % 8 == 0

    mesh = plsc.VectorSubcoreMesh(core_axis_name="c", subcore_axis_name="s")

    @functools.partial(
        pl.kernel,
        mesh=mesh,
        out_type=jax.ShapeDtypeStruct((batch * _FLAT,), jnp.float32),
        scratch_types=[
            pltpu.VMEM((chunk,), jnp.float32),
            pltpu.SemaphoreType.DMA,
        ],
    )
    def tile_broadcast(table_hbm, out_hbm, buf, sem):
        wid = lax.axis_index("c") * info.num_subcores + lax.axis_index("s")
        base = wid * chunk
        pltpu.sync_copy(table_hbm.at[pl.ds(base, chunk)], buf)
        writes = []
        for b in range(batch):
            writes.append(
                pltpu.async_copy(buf, out_hbm.at[pl.ds(b * _FLAT + base, chunk)], sem)
            )
        for w in writes:
            w.wait()

    return tile_broadcast


def kernel(x, table):
    batch = x.shape[0]
    out_flat = _build(batch)(table.reshape(_FLAT))
    return out_flat.reshape(batch, _NUM_QUERIES, _EMBED_DIM)
